# baseline (speedup 1.0000x reference)
import functools

import jax
import jax.numpy as jnp
from jax import lax
from jax.experimental import pallas as pl
from jax.experimental.pallas import tpu as pltpu
from jax.experimental.pallas import tpu_sc as plsc

_R, _B, _D = 256, 32, 1024
_K = 6
_RBLK = 32
_NBLK = _R // _RBLK
_L = 16
_NCHUNK = _R // _L
_NC = 2


def _adj_body(h_ref, nm_ref, w_ref, th_ref, adj_ref, hs0_ref):
    h = h_ref[...]
    nm = nm_ref[...]
    w = w_ref[...]
    s = jnp.dot(h.reshape(_RBLK * _B, _D), w,
                preferred_element_type=jnp.float32).reshape(_RBLK, _B)
    fb = jnp.sqrt(jnp.sum(nm * nm, axis=-1))
    th = th_ref[...]
    adj_ref[...] = (s - th - 0.5 * ((1.0 - 0.9) * fb)).T[None]
    hs0_ref[...] = jnp.zeros((_RBLK, _B, _D), jnp.float32)


def _nms_sc_body(adj_hbm, h_hbm, hs_hbm, hard_hbm,
                 adj_v, sup_v, mask_v, cur_v, idx_buf, rows_v, sem, sem2):
    wid = lax.axis_index("s") * _NC + lax.axis_index("c")
    for k in range(_NBLK):
        pltpu.sync_copy(adj_hbm.at[k, wid], adj_v.at[pl.ds(_RBLK * k, _RBLK)])

    zero = jnp.zeros((_L,), jnp.float32)
    for c in range(_NCHUNK):
        sup_v[pl.ds(_L * c, _L)] = zero
        mask_v[pl.ds(_L * c, _L)] = zero

    iota = lax.iota(jnp.int32, _L)
    neg = jnp.full((_L,), -jnp.inf, jnp.float32)
    offs = jnp.where(iota == 1, 1, jnp.where(iota == 2, _R - 1, 0))
    ones = jnp.ones((_L,), jnp.float32)

    def bcast_max(x):
        return plsc.cummax(lax.rev(plsc.cummax(x), (0,)))

    for t in range(_K):
        acc = neg
        for c in range(_NCHUNK):
            a = adj_v[pl.ds(_L * c, _L)]
            s = sup_v[pl.ds(_L * c, _L)]
            cur = jnp.where(s > 0, neg, a)
            cur_v[pl.ds(_L * c, _L)] = cur
            acc = jnp.maximum(acc, cur)
        m = bcast_max(acc)
        acci = jnp.full((_L,), 2 * _R, jnp.int32)
        for c in range(_NCHUNK):
            cur = cur_v[pl.ds(_L * c, _L)]
            cand = jnp.where(cur == m, iota + _L * c, 2 * _R)
            acci = jnp.minimum(acci, cand)
        idx = -bcast_max(-acci)
        idxvec = (idx + offs) % _R
        plsc.store_scatter(mask_v, [idxvec], ones, mask=iota < 1)
        plsc.store_scatter(sup_v, [idxvec], ones, mask=iota < 3)
        rowvec = idx * _B + wid
        if t == 0:
            plsc.store_scatter(idx_buf, [iota], rowvec, mask=iota < 8)
        else:
            plsc.store_scatter(idx_buf, [iota * 0 + t], rowvec, mask=iota < 1)

    pltpu.sync_copy(mask_v, hard_hbm.at[wid])
    pltpu.async_copy(h_hbm.at[idx_buf], rows_v, sem).wait()
    pltpu.async_copy(rows_v, hs_hbm.at[idx_buf], sem2).wait()


def kernel(H, neighbor_msg, W_score, theta):
    adj3, hs0 = pl.pallas_call(
        _adj_body,
        grid=(_NBLK,),
        in_specs=[
            pl.BlockSpec((_RBLK, _B, _D), lambda i: (i, 0, 0)),
            pl.BlockSpec((_RBLK, _B, _D), lambda i: (i, 0, 0)),
            pl.BlockSpec((_D, 1), lambda i: (0, 0)),
            pl.BlockSpec((_RBLK, 1), lambda i: (i, 0)),
        ],
        out_specs=[
            pl.BlockSpec((1, _B, _RBLK), lambda i: (i, 0, 0)),
            pl.BlockSpec((_RBLK, _B, _D), lambda i: (i, 0, 0)),
        ],
        out_shape=[
            jax.ShapeDtypeStruct((_NBLK, _B, _RBLK), jnp.float32),
            jax.ShapeDtypeStruct((_R, _B, _D), jnp.float32),
        ],
    )(H, neighbor_msg, W_score, theta.reshape(_R, 1))

    hs_ref = jax.new_ref(hs0.reshape(_R * _B, _D))
    nms = functools.partial(
        pl.kernel,
        mesh=plsc.VectorSubcoreMesh(core_axis_name="c", subcore_axis_name="s"),
        out_type=jax.ShapeDtypeStruct((_B, _R), jnp.float32),
        scratch_types=[
            pltpu.VMEM((_R,), jnp.float32),
            pltpu.VMEM((_R,), jnp.float32),
            pltpu.VMEM((_R,), jnp.float32),
            pltpu.VMEM((_R,), jnp.float32),
            pltpu.VMEM((8,), jnp.int32),
            pltpu.VMEM((8, _D), jnp.float32),
            pltpu.SemaphoreType.DMA,
            pltpu.SemaphoreType.DMA,
        ],
        compiler_params=pltpu.CompilerParams(
            needs_layout_passes=False, use_tc_tiling_on_sc=False),
    )(_nms_sc_body)
    hard = nms(adj3, H.reshape(_R * _B, _D), hs_ref)

    Hs = jax.freeze(hs_ref).reshape(_R, _B, _D)
    adj = adj3.transpose(1, 0, 2).reshape(_B, _R)
    return (Hs, hard, adj)


# speedup vs baseline: 1.0011x; 1.0011x over previous
import functools

import jax
import jax.numpy as jnp
from jax import lax
from jax.experimental import pallas as pl
from jax.experimental.pallas import tpu as pltpu
from jax.experimental.pallas import tpu_sc as plsc

_R, _B, _D = 256, 32, 1024
_K = 6
_RBLK = 32
_NBLK = _R // _RBLK
_L = 16
_NCHUNK = _R // _L
_NC = 2


def _adj_body(h_ref, nm_ref, w_ref, th_ref, adj_ref, hs0_ref):
    h = h_ref[...]
    nm = nm_ref[...]
    w = w_ref[...]
    s = jnp.dot(h.reshape(_RBLK * _B, _D), w,
                preferred_element_type=jnp.float32).reshape(_RBLK, _B)
    fb = jnp.sqrt(jnp.sum(nm * nm, axis=-1))
    th = th_ref[...]
    adj_ref[...] = (s - th - 0.5 * ((1.0 - 0.9) * fb)).T[None]
    hs0_ref[...] = jnp.zeros((_RBLK * _B, _D), jnp.float32)


def _nms_sc_body(adj_hbm, h_hbm, hs_hbm, hard_hbm,
                 adj_v, sup_v, mask_v, cur_v, idx_buf, rows_v, sem, sem2):
    wid = lax.axis_index("s") * _NC + lax.axis_index("c")
    for k in range(_NBLK):
        pltpu.sync_copy(adj_hbm.at[k, wid], adj_v.at[pl.ds(_RBLK * k, _RBLK)])

    zero = jnp.zeros((_L,), jnp.float32)
    for c in range(_NCHUNK):
        sup_v[pl.ds(_L * c, _L)] = zero
        mask_v[pl.ds(_L * c, _L)] = zero

    iota = lax.iota(jnp.int32, _L)
    neg = jnp.full((_L,), -jnp.inf, jnp.float32)
    offs = jnp.where(iota == 1, 1, jnp.where(iota == 2, _R - 1, 0))
    ones = jnp.ones((_L,), jnp.float32)

    def bcast_max(x):
        return plsc.cummax(lax.rev(plsc.cummax(x), (0,)))

    for t in range(_K):
        acc = neg
        for c in range(_NCHUNK):
            a = adj_v[pl.ds(_L * c, _L)]
            s = sup_v[pl.ds(_L * c, _L)]
            cur = jnp.where(s > 0, neg, a)
            cur_v[pl.ds(_L * c, _L)] = cur
            acc = jnp.maximum(acc, cur)
        m = bcast_max(acc)
        acci = jnp.full((_L,), 2 * _R, jnp.int32)
        for c in range(_NCHUNK):
            cur = cur_v[pl.ds(_L * c, _L)]
            cand = jnp.where(cur == m, iota + _L * c, 2 * _R)
            acci = jnp.minimum(acci, cand)
        idx = -bcast_max(-acci)
        idxvec = (idx + offs) % _R
        plsc.store_scatter(mask_v, [idxvec], ones, mask=iota < 1)
        plsc.store_scatter(sup_v, [idxvec], ones, mask=iota < 3)
        rowvec = idx * _B + wid
        if t == 0:
            plsc.store_scatter(idx_buf, [iota], rowvec, mask=iota < 8)
        else:
            plsc.store_scatter(idx_buf, [iota * 0 + t], rowvec, mask=iota < 1)

    pltpu.sync_copy(mask_v, hard_hbm.at[wid])
    pltpu.async_copy(h_hbm.at[idx_buf], rows_v, sem).wait()
    pltpu.async_copy(rows_v, hs_hbm.at[idx_buf], sem2).wait()


def kernel(H, neighbor_msg, W_score, theta):
    adj3, hs0 = pl.pallas_call(
        _adj_body,
        grid=(_NBLK,),
        in_specs=[
            pl.BlockSpec((_RBLK, _B, _D), lambda i: (i, 0, 0)),
            pl.BlockSpec((_RBLK, _B, _D), lambda i: (i, 0, 0)),
            pl.BlockSpec((_D, 1), lambda i: (0, 0)),
            pl.BlockSpec((_RBLK, 1), lambda i: (i, 0)),
        ],
        out_specs=[
            pl.BlockSpec((1, _B, _RBLK), lambda i: (i, 0, 0)),
            pl.BlockSpec((_RBLK * _B, _D), lambda i: (i, 0)),
        ],
        out_shape=[
            jax.ShapeDtypeStruct((_NBLK, _B, _RBLK), jnp.float32),
            jax.ShapeDtypeStruct((_R * _B, _D), jnp.float32),
        ],
    )(H, neighbor_msg, W_score, theta.reshape(_R, 1))

    hs_ref = jax.new_ref(hs0)
    nms = functools.partial(
        pl.kernel,
        mesh=plsc.VectorSubcoreMesh(core_axis_name="c", subcore_axis_name="s"),
        out_type=jax.ShapeDtypeStruct((_B, _R), jnp.float32),
        scratch_types=[
            pltpu.VMEM((_R,), jnp.float32),
            pltpu.VMEM((_R,), jnp.float32),
            pltpu.VMEM((_R,), jnp.float32),
            pltpu.VMEM((_R,), jnp.float32),
            pltpu.VMEM((8,), jnp.int32),
            pltpu.VMEM((8, _D), jnp.float32),
            pltpu.SemaphoreType.DMA,
            pltpu.SemaphoreType.DMA,
        ],
        compiler_params=pltpu.CompilerParams(
            needs_layout_passes=False, use_tc_tiling_on_sc=False),
    )(_nms_sc_body)
    hard = nms(adj3, H.reshape(_R * _B, _D), hs_ref)

    Hs = jax.freeze(hs_ref).reshape(_R, _B, _D)
    adj = adj3.transpose(1, 0, 2).reshape(_B, _R)
    return (Hs, hard, adj)


# speedup vs baseline: 1.0015x; 1.0005x over previous
import functools

import jax
import jax.numpy as jnp
from jax import lax
from jax.experimental import pallas as pl
from jax.experimental.pallas import tpu as pltpu
from jax.experimental.pallas import tpu_sc as plsc
from jax._src.pallas import mpmd as _mpmd

_R, _B, _D = 256, 32, 1024
_K = 6
_RBLK = 32
_NBLK = _R // _RBLK
_L = 16
_NCHUNK = _R // _L
_NC = 2


def _adj_body(h_ref, nm_ref, w_ref, th_ref, adj_ref, hs0_ref):
    h = h_ref[...]
    nm = nm_ref[...]
    w = w_ref[...]
    s = jnp.dot(h.reshape(_RBLK * _B, _D), w,
                preferred_element_type=jnp.float32).reshape(_RBLK, _B)
    fb = jnp.sqrt(jnp.sum(nm * nm, axis=-1))
    th = th_ref[...]
    adj_ref[...] = (s - th - 0.5 * ((1.0 - 0.9) * fb)).T[None]
    hs0_ref[...] = jnp.zeros((_RBLK * _B, _D), jnp.float32)


def _nms_sc_body(adj_hbm, h_hbm, hs0_hbm, hard_hbm, hs_hbm,
                 adj_v, sup_v, mask_v, cur_v, idx_buf, rows_v, sem, sem2):
    del hs0_hbm
    wid = lax.axis_index("s") * _NC + lax.axis_index("c")
    for k in range(_NBLK):
        pltpu.sync_copy(adj_hbm.at[k, wid], adj_v.at[pl.ds(_RBLK * k, _RBLK)])

    zero = jnp.zeros((_L,), jnp.float32)
    for c in range(_NCHUNK):
        sup_v[pl.ds(_L * c, _L)] = zero
        mask_v[pl.ds(_L * c, _L)] = zero

    iota = lax.iota(jnp.int32, _L)
    neg = jnp.full((_L,), -jnp.inf, jnp.float32)
    offs = jnp.where(iota == 1, 1, jnp.where(iota == 2, _R - 1, 0))
    ones = jnp.ones((_L,), jnp.float32)

    def bcast_max(x):
        return plsc.cummax(lax.rev(plsc.cummax(x), (0,)))

    for t in range(_K):
        acc = neg
        for c in range(_NCHUNK):
            a = adj_v[pl.ds(_L * c, _L)]
            s = sup_v[pl.ds(_L * c, _L)]
            cur = jnp.where(s > 0, neg, a)
            cur_v[pl.ds(_L * c, _L)] = cur
            acc = jnp.maximum(acc, cur)
        m = bcast_max(acc)
        acci = jnp.full((_L,), 2 * _R, jnp.int32)
        for c in range(_NCHUNK):
            cur = cur_v[pl.ds(_L * c, _L)]
            cand = jnp.where(cur == m, iota + _L * c, 2 * _R)
            acci = jnp.minimum(acci, cand)
        idx = -bcast_max(-acci)
        idxvec = (idx + offs) % _R
        plsc.store_scatter(mask_v, [idxvec], ones, mask=iota < 1)
        plsc.store_scatter(sup_v, [idxvec], ones, mask=iota < 3)
        rowvec = idx * _B + wid
        if t == 0:
            plsc.store_scatter(idx_buf, [iota], rowvec, mask=iota < 8)
        else:
            plsc.store_scatter(idx_buf, [iota * 0 + t], rowvec, mask=iota < 1)

    pltpu.sync_copy(mask_v, hard_hbm.at[wid])
    pltpu.async_copy(h_hbm.at[idx_buf], rows_v, sem).wait()
    pltpu.async_copy(rows_v, hs_hbm.at[idx_buf], sem2).wait()


def kernel(H, neighbor_msg, W_score, theta):
    adj3, hs0 = pl.pallas_call(
        _adj_body,
        grid=(_NBLK,),
        in_specs=[
            pl.BlockSpec((_RBLK, _B, _D), lambda i: (i, 0, 0)),
            pl.BlockSpec((_RBLK, _B, _D), lambda i: (i, 0, 0)),
            pl.BlockSpec((_D, 1), lambda i: (0, 0)),
            pl.BlockSpec((_RBLK, 1), lambda i: (i, 0)),
        ],
        out_specs=[
            pl.BlockSpec((1, _B, _RBLK), lambda i: (i, 0, 0)),
            pl.BlockSpec((_RBLK * _B, _D), lambda i: (i, 0)),
        ],
        out_shape=[
            jax.ShapeDtypeStruct((_NBLK, _B, _RBLK), jnp.float32),
            jax.ShapeDtypeStruct((_R * _B, _D), jnp.float32),
        ],
    )(H, neighbor_msg, W_score, theta.reshape(_R, 1))

    mesh = plsc.VectorSubcoreMesh(core_axis_name="c", subcore_axis_name="s")
    nms = _mpmd._mpmd_map(
        [(mesh, _nms_sc_body)],
        out_types=(
            jax.ShapeDtypeStruct((_B, _R), jnp.float32),
            jax.ShapeDtypeStruct((_R * _B, _D), jnp.float32),
        ),
        input_output_aliases={2: 1},
        scratch_types=[
            pltpu.VMEM((_R,), jnp.float32),
            pltpu.VMEM((_R,), jnp.float32),
            pltpu.VMEM((_R,), jnp.float32),
            pltpu.VMEM((_R,), jnp.float32),
            pltpu.VMEM((8,), jnp.int32),
            pltpu.VMEM((8, _D), jnp.float32),
            pltpu.SemaphoreType.DMA,
            pltpu.SemaphoreType.DMA,
        ],
        compiler_params=pltpu.CompilerParams(
            needs_layout_passes=False, use_tc_tiling_on_sc=False),
    )
    hard, hs = nms(adj3, H.reshape(_R * _B, _D), hs0)

    Hs = hs.reshape(_R, _B, _D)
    adj = adj3.transpose(1, 0, 2).reshape(_B, _R)
    return (Hs, hard, adj)


# speedup vs baseline: 2.1541x; 2.1508x over previous
import functools

import jax
import jax.numpy as jnp
from jax import lax
from jax.experimental import pallas as pl
from jax.experimental.pallas import tpu as pltpu
from jax.experimental.pallas import tpu_sc as plsc

_R, _B, _D = 256, 32, 1024
_K = 6
_RBLK = 32
_NBLK = _R // _RBLK
_L = 16
_NCHUNK = _R // _L
_NC = 2


def _adj_body(h_ref, nm_ref, w_ref, th_ref, adj_ref):
    h = h_ref[...]
    nm = nm_ref[...]
    w = w_ref[...]
    s = jnp.dot(h.reshape(_RBLK * _B, _D), w,
                preferred_element_type=jnp.float32).reshape(_RBLK, _B)
    fb = jnp.sqrt(jnp.sum(nm * nm, axis=-1))
    th = th_ref[...]
    adj_ref[...] = s - th - 0.5 * ((1.0 - 0.9) * fb)


def _nms_sc_body(adj_hbm, hard_hbm, adj_v, sup_v, mask_v, cur_v):
    wid = lax.axis_index("s") * _NC + lax.axis_index("c")
    pltpu.sync_copy(adj_hbm.at[wid], adj_v)

    zero = jnp.zeros((_L,), jnp.float32)
    for c in range(_NCHUNK):
        sup_v[pl.ds(_L * c, _L)] = zero
        mask_v[pl.ds(_L * c, _L)] = zero

    iota = lax.iota(jnp.int32, _L)
    neg = jnp.full((_L,), -jnp.inf, jnp.float32)
    offs = jnp.where(iota == 1, 1, jnp.where(iota == 2, _R - 1, 0))
    ones = jnp.ones((_L,), jnp.float32)

    def bcast_max(x):
        return plsc.cummax(lax.rev(plsc.cummax(x), (0,)))

    for _ in range(_K):
        acc = neg
        for c in range(_NCHUNK):
            a = adj_v[pl.ds(_L * c, _L)]
            s = sup_v[pl.ds(_L * c, _L)]
            cur = jnp.where(s > 0, neg, a)
            cur_v[pl.ds(_L * c, _L)] = cur
            acc = jnp.maximum(acc, cur)
        m = bcast_max(acc)
        acci = jnp.full((_L,), 2 * _R, jnp.int32)
        for c in range(_NCHUNK):
            cur = cur_v[pl.ds(_L * c, _L)]
            cand = jnp.where(cur == m, iota + _L * c, 2 * _R)
            acci = jnp.minimum(acci, cand)
        idx = -bcast_max(-acci)
        idxvec = (idx + offs) % _R
        plsc.store_scatter(mask_v, [idxvec], ones, mask=iota < 1)
        plsc.store_scatter(sup_v, [idxvec], ones, mask=iota < 3)

    pltpu.sync_copy(mask_v, hard_hbm.at[wid])


def _scale_body(h_ref, m_ref, out_ref):
    out_ref[...] = h_ref[...] * m_ref[...][:, :, None]


def kernel(H, neighbor_msg, W_score, theta):
    adj_t = pl.pallas_call(
        _adj_body,
        grid=(_NBLK,),
        in_specs=[
            pl.BlockSpec((_RBLK, _B, _D), lambda i: (i, 0, 0)),
            pl.BlockSpec((_RBLK, _B, _D), lambda i: (i, 0, 0)),
            pl.BlockSpec((_D, 1), lambda i: (0, 0)),
            pl.BlockSpec((_RBLK, 1), lambda i: (i, 0)),
        ],
        out_specs=pl.BlockSpec((_RBLK, _B), lambda i: (i, 0)),
        out_shape=jax.ShapeDtypeStruct((_R, _B), jnp.float32),
    )(H, neighbor_msg, W_score, theta.reshape(_R, 1))

    adj = adj_t.T

    nms = functools.partial(
        pl.kernel,
        mesh=plsc.VectorSubcoreMesh(core_axis_name="c", subcore_axis_name="s"),
        out_type=jax.ShapeDtypeStruct((_B, _R), jnp.float32),
        scratch_types=[pltpu.VMEM((_R,), jnp.float32)] * 4,
        compiler_params=pltpu.CompilerParams(
            needs_layout_passes=False, use_tc_tiling_on_sc=False,
            skip_device_barrier=True),
    )(_nms_sc_body)
    hard = nms(adj)

    Hs = pl.pallas_call(
        _scale_body,
        grid=(_NBLK,),
        in_specs=[
            pl.BlockSpec((_RBLK, _B, _D), lambda i: (i, 0, 0)),
            pl.BlockSpec((_RBLK, _B), lambda i: (i, 0)),
        ],
        out_specs=pl.BlockSpec((_RBLK, _B, _D), lambda i: (i, 0, 0)),
        out_shape=jax.ShapeDtypeStruct((_R, _B, _D), jnp.float32),
    )(H, hard.T)

    return (Hs, hard, adj)
